# baseline (speedup 1.0000x reference)
import functools

import jax
import jax.numpy as jnp
from jax import lax
from jax.experimental import pallas as pl
from jax.experimental.pallas import tpu as pltpu
from jax.experimental.pallas import tpu_sc as plsc

B = 16384
K = 64
L = 16
IDX_CHUNK = 128


def _make_kernel(num_cores, num_subcores):
    nw = num_cores * num_subcores
    bpw = B // nw
    n_chunks = bpw // IDX_CHUNK

    mesh = plsc.VectorSubcoreMesh(core_axis_name="c", subcore_axis_name="s")

    @functools.partial(
        pl.kernel,
        mesh=mesh,
        compiler_params=pltpu.CompilerParams(
            needs_layout_passes=False, use_tc_tiling_on_sc=False),
        out_type=jax.ShapeDtypeStruct((B,), jnp.float32),
        scratch_types=[
            pltpu.VMEM((n_chunks, IDX_CHUNK), jnp.int32),
            pltpu.VMEM((n_chunks, IDX_CHUNK), jnp.int32),
            pltpu.VMEM((bpw, K), jnp.float32),
            pltpu.VMEM((bpw, K), jnp.float32),
            pltpu.VMEM((bpw,), jnp.float32),
            pltpu.SemaphoreType.DMA,
        ],
    )
    def mf_kernel(user_hbm, item_hbm, uemb_hbm, iemb_hbm, out_hbm,
                  idx_u, idx_i, rows_u, rows_i, out_v, sem):
        cid = lax.axis_index("c")
        sid = lax.axis_index("s")
        wid = sid * num_cores + cid
        base = wid * bpw

        for j in range(n_chunks):
            pltpu.sync_copy(user_hbm.at[pl.ds(base + j * IDX_CHUNK, IDX_CHUNK)],
                            idx_u.at[j])
            pltpu.sync_copy(item_hbm.at[pl.ds(base + j * IDX_CHUNK, IDX_CHUNK)],
                            idx_i.at[j])

        copies = []
        for j in range(n_chunks):
            copies.append(pltpu.async_copy(
                uemb_hbm.at[idx_u.at[j]],
                rows_u.at[pl.ds(j * IDX_CHUNK, IDX_CHUNK)], sem))
            copies.append(pltpu.async_copy(
                iemb_hbm.at[idx_i.at[j]],
                rows_i.at[pl.ds(j * IDX_CHUNK, IDX_CHUNK)], sem))
        for c in copies:
            c.wait()

        lanes = lax.iota(jnp.int32, L)

        def gbody(g, carry):
            rows = lanes + g * L
            acc = jnp.zeros((L,), jnp.float32)
            for k in range(K):
                col = jnp.full((L,), k, jnp.int32)
                uk = plsc.load_gather(rows_u, [rows, col])
                vk = plsc.load_gather(rows_i, [rows, col])
                acc = acc + uk * vk
            plsc.store_scatter(out_v, [rows], acc)
            return carry

        lax.fori_loop(0, bpw // L, gbody, 0)

        pltpu.sync_copy(out_v, out_hbm.at[pl.ds(base, bpw)])

    return mf_kernel


def kernel(user, item, users_emb, items_emb):
    info = plsc.get_sparse_core_info()
    f = _make_kernel(info.num_cores, info.num_subcores)
    return f(user, item, users_emb, items_emb)
